# Initial kernel scaffold; baseline (speedup 1.0000x reference)
import jax
import jax.numpy as jnp
from jax.experimental import pallas as pl


def kernel(cur_f, pre_h, point, Wz, bz, Wr, br, Ws, bs, Wfc, bfc):
    raise NotImplementedError("write your pallas kernel here")


# speedup vs baseline: 9.4128x; 9.4128x over previous
import functools

import jax
import jax.numpy as jnp
from jax import lax
from jax.experimental import pallas as pl
from jax.experimental.pallas import tpu as pltpu
from jax.experimental.pallas import tpu_sc as plsc

_K = 32
_R2 = 0.25
_TN = 512
_TM = 512


def _ball_query_body(pq_ref, pt_ref, idx_ref):
    b = pl.program_id(0)
    n_all = pt_ref.shape[2]
    q = pq_ref[0]
    qx = q[:, 0:1]
    qy = q[:, 1:2]
    qz = q[:, 2:3]

    ri = lax.broadcasted_iota(jnp.int32, (_TM, _TM), 0)
    ci = lax.broadcasted_iota(jnp.int32, (_TM, _TM), 1)
    lt = (ri < ci).astype(jnp.float32)
    koh = lax.broadcasted_iota(jnp.int32, (1, _K), 1)

    def mstep(t, carry):
        cnt, acc = carry
        m0 = t * _TM
        px = pt_ref[0, 0:1, pl.ds(m0, _TM)]
        py = pt_ref[0, 1:2, pl.ds(m0, _TM)]
        pz = pt_ref[0, 2:3, pl.ds(m0, _TM)]
        dx = qx - px
        dy = qy - py
        dz = qz - pz
        d2 = (dx * dx + dy * dy) + dz * dz
        mf = (d2 < _R2).astype(jnp.float32)
        exr = lax.dot(mf, lt, preferred_element_type=jnp.float32)
        s = cnt + exr
        validf = mf * (s < _K).astype(jnp.float32)
        mvals = (m0 + lax.broadcasted_iota(jnp.int32, (1, _TM), 1)).astype(
            jnp.float32)
        for k in range(_K):
            eq = jnp.where(s == float(k), validf, 0.0)
            contrib = jnp.sum(eq * mvals, axis=1, keepdims=True)
            oh = (koh == k).astype(jnp.float32)
            acc = acc + contrib * oh
        cnt = cnt + jnp.sum(mf, axis=1, keepdims=True)
        return cnt, acc

    cnt0 = jnp.zeros((_TN, 1), jnp.float32)
    acc0 = jnp.zeros((_TN, _K), jnp.float32)
    cnt, acc = lax.fori_loop(0, n_all // _TM, mstep, (cnt0, acc0))

    kio = lax.broadcasted_iota(jnp.int32, (_TN, _K), 1).astype(jnp.float32)
    first = acc[:, 0:1]
    idxf = jnp.where(kio < cnt, acc, first)
    idx_ref[0] = idxf.astype(jnp.int32) + b * n_all


def _dgT(a, w):
    return lax.dot_general(a, w, (((1,), (1,)), ((), ())),
                           preferred_element_type=jnp.float32)


def _uv_body(phT_ref, cfT_ref, pq_ref, wz_ref, wr_ref, ws_ref,
             bz_ref, br_ref, bs_ref, u_ref, v_ref):
    c = wz_ref.shape[0]
    ph = phT_ref[0]
    cf = cfT_ref[0]
    pt = pq_ref[0]
    us = []
    vs = []
    for w_ref, b_ref, has_cf in ((wz_ref, bz_ref, True),
                                 (wr_ref, br_ref, True),
                                 (ws_ref, bs_ref, False)):
        w = w_ref[...]
        w1 = w[:, :c]
        w3 = w[:, w.shape[1] - 3:]
        pw = _dgT(pt, w3)
        u = _dgT(ph, w1) + pw
        bias = b_ref[...].reshape(1, c)
        if has_cf:
            v = _dgT(cf, w[:, c:2 * c]) - pw + bias
        else:
            v = bias - pw
        us.append(u)
        vs.append(v)
    u_ref[0] = jnp.concatenate(us, axis=1)
    v_ref[0] = jnp.concatenate(vs, axis=1)


def _final_body(g_ref, v_ref, phT_ref, cfT_ref, wfc_ref, bfc_ref, out_ref):
    c = wfc_ref.shape[0]
    g = jnp.maximum(g_ref[0] + v_ref[0], 0.0)
    z = jax.nn.sigmoid(g[:, :c])
    r = jax.nn.sigmoid(g[:, c:2 * c])
    s_old = g[:, 2 * c:]
    ph = phT_ref[0]
    cf = cfT_ref[0]
    w = wfc_ref[...]
    s_new = jnp.tanh(_dgT(ph, w[:, :c]) + _dgT(r * cf, w[:, c:])
                     + bfc_ref[...].reshape(1, c))
    out_ref[0] = z * s_old + (1.0 - z) * s_new


def _make_sc_gather_max(rows, width, k):
    ncores, nsub = 2, 16
    nw = ncores * nsub
    pts = rows // nw
    lanes = 16
    mesh = plsc.VectorSubcoreMesh(core_axis_name="c", subcore_axis_name="s")

    @functools.partial(
        pl.kernel, mesh=mesh,
        out_type=jax.ShapeDtypeStruct((rows, width), jnp.float32),
        scratch_types=[
            pltpu.VMEM((pts, k), jnp.int32),
            pltpu.VMEM((k, width), jnp.float32),
            pltpu.VMEM((1, width), jnp.float32),
            pltpu.SemaphoreType.DMA,
        ],
    )
    def sc_gather_max(table_hbm, idx_hbm, out_hbm, idx_v, rows_v, acc_v, sem):
        wid = lax.axis_index("s") * ncores + lax.axis_index("c")
        base = wid * pts
        pltpu.sync_copy(idx_hbm.at[pl.ds(base, pts)], idx_v)

        def point_body(p, carry):
            pltpu.async_copy(table_hbm.at[idx_v.at[p]], rows_v, sem).wait()

            def col_body(j, inner):
                o = j * lanes
                a = rows_v[0, pl.ds(o, lanes)]
                for kk in range(1, k):
                    a = jnp.maximum(a, rows_v[kk, pl.ds(o, lanes)])
                acc_v[0, pl.ds(o, lanes)] = a
                return inner

            lax.fori_loop(0, width // lanes, col_body, 0)
            pltpu.sync_copy(acc_v, out_hbm.at[pl.ds(base + p, 1)])
            return carry

        lax.fori_loop(0, pts, point_body, 0)

    return sc_gather_max


def kernel(cur_f, pre_h, point, Wz, bz, Wr, br, Ws, bs, Wfc, bfc):
    b, c, n = cur_f.shape
    k = _K
    pT = jnp.transpose(point, (0, 2, 1))
    phT = jnp.transpose(pre_h, (0, 2, 1))
    cfT = jnp.transpose(cur_f, (0, 2, 1))
    grid = (b, n // _TN)

    idx = pl.pallas_call(
        _ball_query_body,
        grid=grid,
        in_specs=[
            pl.BlockSpec((1, _TN, 3), lambda bi, i: (bi, i, 0)),
            pl.BlockSpec((1, 3, n), lambda bi, i: (bi, 0, 0)),
        ],
        out_specs=pl.BlockSpec((1, _TN, k), lambda bi, i: (bi, i, 0)),
        out_shape=jax.ShapeDtypeStruct((b, n, k), jnp.int32),
    )(point, pT)

    wfull = lambda shape: pl.BlockSpec(shape, lambda bi, i: tuple(
        0 for _ in shape))
    u, v = pl.pallas_call(
        _uv_body,
        grid=grid,
        in_specs=[
            pl.BlockSpec((1, _TN, c), lambda bi, i: (bi, i, 0)),
            pl.BlockSpec((1, _TN, c), lambda bi, i: (bi, i, 0)),
            pl.BlockSpec((1, _TN, 3), lambda bi, i: (bi, i, 0)),
            wfull(Wz.shape), wfull(Wr.shape), wfull(Ws.shape),
            wfull(bz.shape), wfull(br.shape), wfull(bs.shape),
        ],
        out_specs=[
            pl.BlockSpec((1, _TN, 3 * c), lambda bi, i: (bi, i, 0)),
            pl.BlockSpec((1, _TN, 3 * c), lambda bi, i: (bi, i, 0)),
        ],
        out_shape=[
            jax.ShapeDtypeStruct((b, n, 3 * c), jnp.float32),
            jax.ShapeDtypeStruct((b, n, 3 * c), jnp.float32),
        ],
    )(phT, cfT, point, Wz, Wr, Ws, bz, br, bs)

    sc = _make_sc_gather_max(b * n, 3 * c, k)
    gmax = sc(u.reshape(b * n, 3 * c), idx.reshape(b * n, k))
    gmax = gmax.reshape(b, n, 3 * c)

    s1t = pl.pallas_call(
        _final_body,
        grid=grid,
        in_specs=[
            pl.BlockSpec((1, _TN, 3 * c), lambda bi, i: (bi, i, 0)),
            pl.BlockSpec((1, _TN, 3 * c), lambda bi, i: (bi, i, 0)),
            pl.BlockSpec((1, _TN, c), lambda bi, i: (bi, i, 0)),
            pl.BlockSpec((1, _TN, c), lambda bi, i: (bi, i, 0)),
            wfull(Wfc.shape), wfull(bfc.shape),
        ],
        out_specs=pl.BlockSpec((1, _TN, c), lambda bi, i: (bi, i, 0)),
        out_shape=jax.ShapeDtypeStruct((b, n, c), jnp.float32),
    )(gmax, v, phT, cfT, Wfc, bfc)

    s1 = jnp.transpose(s1t, (0, 2, 1))
    return (s1, s1)


# speedup vs baseline: 11.5143x; 1.2233x over previous
import functools

import jax
import jax.numpy as jnp
from jax import lax
from jax.experimental import pallas as pl
from jax.experimental.pallas import tpu as pltpu
from jax.experimental.pallas import tpu_sc as plsc

_K = 32
_R2 = 0.25
_TN = 512
_TM = 512


def _ball_query_body(pq_ref, pt_ref, idx_ref):
    b = pl.program_id(0)
    n_all = pt_ref.shape[2]
    q = pq_ref[0]
    qx = q[:, 0:1]
    qy = q[:, 1:2]
    qz = q[:, 2:3]

    ri = lax.broadcasted_iota(jnp.int32, (_TM, _TM), 0)
    ci = lax.broadcasted_iota(jnp.int32, (_TM, _TM), 1)
    lt = (ri < ci).astype(jnp.float32)
    koh = lax.broadcasted_iota(jnp.int32, (1, _K), 1)

    def mstep(t, carry):
        cnt, acc = carry
        m0 = t * _TM
        px = pt_ref[0, 0:1, pl.ds(m0, _TM)]
        py = pt_ref[0, 1:2, pl.ds(m0, _TM)]
        pz = pt_ref[0, 2:3, pl.ds(m0, _TM)]
        dx = qx - px
        dy = qy - py
        dz = qz - pz
        d2 = (dx * dx + dy * dy) + dz * dz
        mf = (d2 < _R2).astype(jnp.float32)
        exr = lax.dot(mf, lt, preferred_element_type=jnp.float32)
        s = cnt + exr
        validf = mf * (s < _K).astype(jnp.float32)
        mvals = (m0 + lax.broadcasted_iota(jnp.int32, (1, _TM), 1)).astype(
            jnp.float32)
        for k in range(_K):
            eq = jnp.where(s == float(k), validf, 0.0)
            contrib = jnp.sum(eq * mvals, axis=1, keepdims=True)
            oh = (koh == k).astype(jnp.float32)
            acc = acc + contrib * oh
        cnt = cnt + jnp.sum(mf, axis=1, keepdims=True)
        return cnt, acc

    cnt0 = jnp.zeros((_TN, 1), jnp.float32)
    acc0 = jnp.zeros((_TN, _K), jnp.float32)
    cnt, acc = lax.fori_loop(0, n_all // _TM, mstep, (cnt0, acc0))

    kio = lax.broadcasted_iota(jnp.int32, (_TN, _K), 1).astype(jnp.float32)
    first = acc[:, 0:1]
    idxf = jnp.where(kio < cnt, acc, first)
    idx_ref[0] = idxf.astype(jnp.int32) + b * n_all


def _dgT(a, w):
    return lax.dot_general(a, w, (((1,), (1,)), ((), ())),
                           preferred_element_type=jnp.float32)


def _uv_body(phT_ref, cfT_ref, pq_ref, wz_ref, wr_ref, ws_ref,
             bz_ref, br_ref, bs_ref, u_ref, v_ref):
    c = wz_ref.shape[0]
    ph = phT_ref[0]
    cf = cfT_ref[0]
    pt = pq_ref[0]
    us = []
    vs = []
    for w_ref, b_ref, has_cf in ((wz_ref, bz_ref, True),
                                 (wr_ref, br_ref, True),
                                 (ws_ref, bs_ref, False)):
        w = w_ref[...]
        w1 = w[:, :c]
        w3 = w[:, w.shape[1] - 3:]
        pw = _dgT(pt, w3)
        u = _dgT(ph, w1) + pw
        bias = b_ref[...].reshape(1, c)
        if has_cf:
            v = _dgT(cf, w[:, c:2 * c]) - pw + bias
        else:
            v = bias - pw
        us.append(u)
        vs.append(v)
    u_ref[0] = jnp.concatenate(us, axis=1)
    v_ref[0] = jnp.concatenate(vs, axis=1)


def _final_body(g_ref, v_ref, phT_ref, cfT_ref, wfc_ref, bfc_ref, out_ref):
    c = wfc_ref.shape[0]
    g = jnp.maximum(g_ref[0] + v_ref[0], 0.0)
    z = jax.nn.sigmoid(g[:, :c])
    r = jax.nn.sigmoid(g[:, c:2 * c])
    s_old = g[:, 2 * c:]
    ph = phT_ref[0]
    cf = cfT_ref[0]
    w = wfc_ref[...]
    s_new = jnp.tanh(_dgT(ph, w[:, :c]) + _dgT(r * cf, w[:, c:])
                     + bfc_ref[...].reshape(1, c))
    out_ref[0] = z * s_old + (1.0 - z) * s_new


def _make_sc_gather_max(rows, width, k):
    ncores, nsub = 2, 16
    nw = ncores * nsub
    pts = rows // nw
    lanes = 16
    mesh = plsc.VectorSubcoreMesh(core_axis_name="c", subcore_axis_name="s")

    pairs = pts // 2

    @functools.partial(
        pl.kernel, mesh=mesh,
        out_type=jax.ShapeDtypeStruct((rows, width), jnp.float32),
        scratch_types=[
            pltpu.VMEM((pts, k), jnp.int32),
            pltpu.VMEM((k, width), jnp.float32),
            pltpu.VMEM((k, width), jnp.float32),
            pltpu.VMEM((1, width), jnp.float32),
            pltpu.VMEM((1, width), jnp.float32),
            pltpu.SemaphoreType.DMA,
            pltpu.SemaphoreType.DMA,
            pltpu.SemaphoreType.DMA,
            pltpu.SemaphoreType.DMA,
        ],
    )
    def sc_gather_max(table_hbm, idx_hbm, out_hbm, idx_v, rows_a, rows_b,
                      acc_a, acc_b, sem_a, sem_b, sem_oa, sem_ob):
        wid = lax.axis_index("s") * ncores + lax.axis_index("c")
        base = wid * pts
        pltpu.sync_copy(idx_hbm.at[pl.ds(base, pts)], idx_v)

        def reduce_into(rows_v, acc_v):
            def col_body(j, inner):
                o = j * lanes
                a = rows_v[0, pl.ds(o, lanes)]
                for kk in range(1, k):
                    a = jnp.maximum(a, rows_v[kk, pl.ds(o, lanes)])
                acc_v[0, pl.ds(o, lanes)] = a
                return inner

            lax.fori_loop(0, width // lanes, col_body, 0)

        pltpu.async_copy(table_hbm.at[idx_v.at[0]], rows_a, sem_a)

        def pair_body(p2, carry):
            p = 2 * p2
            pltpu.async_copy(table_hbm.at[idx_v.at[p + 1]], rows_b, sem_b)
            pltpu.make_async_copy(table_hbm.at[idx_v.at[p]], rows_a,
                                  sem_a).wait()

            @pl.when(p2 > 0)
            def _():
                pltpu.make_async_copy(acc_a, out_hbm.at[pl.ds(base, 1)],
                                      sem_oa).wait()

            reduce_into(rows_a, acc_a)
            pltpu.async_copy(acc_a, out_hbm.at[pl.ds(base + p, 1)], sem_oa)

            @pl.when(p2 < pairs - 1)
            def _():
                pltpu.async_copy(table_hbm.at[idx_v.at[p + 2]], rows_a, sem_a)

            pltpu.make_async_copy(table_hbm.at[idx_v.at[p + 1]], rows_b,
                                  sem_b).wait()

            @pl.when(p2 > 0)
            def _():
                pltpu.make_async_copy(acc_b, out_hbm.at[pl.ds(base, 1)],
                                      sem_ob).wait()

            reduce_into(rows_b, acc_b)
            pltpu.async_copy(acc_b, out_hbm.at[pl.ds(base + p + 1, 1)], sem_ob)
            return carry

        lax.fori_loop(0, pairs, pair_body, 0)
        pltpu.make_async_copy(acc_a, out_hbm.at[pl.ds(base, 1)], sem_oa).wait()
        pltpu.make_async_copy(acc_b, out_hbm.at[pl.ds(base, 1)], sem_ob).wait()

    return sc_gather_max


def kernel(cur_f, pre_h, point, Wz, bz, Wr, br, Ws, bs, Wfc, bfc):
    b, c, n = cur_f.shape
    k = _K
    pT = jnp.transpose(point, (0, 2, 1))
    phT = jnp.transpose(pre_h, (0, 2, 1))
    cfT = jnp.transpose(cur_f, (0, 2, 1))
    grid = (b, n // _TN)

    idx = pl.pallas_call(
        _ball_query_body,
        grid=grid,
        in_specs=[
            pl.BlockSpec((1, _TN, 3), lambda bi, i: (bi, i, 0)),
            pl.BlockSpec((1, 3, n), lambda bi, i: (bi, 0, 0)),
        ],
        out_specs=pl.BlockSpec((1, _TN, k), lambda bi, i: (bi, i, 0)),
        out_shape=jax.ShapeDtypeStruct((b, n, k), jnp.int32),
    )(point, pT)

    wfull = lambda shape: pl.BlockSpec(shape, lambda bi, i: tuple(
        0 for _ in shape))
    u, v = pl.pallas_call(
        _uv_body,
        grid=grid,
        in_specs=[
            pl.BlockSpec((1, _TN, c), lambda bi, i: (bi, i, 0)),
            pl.BlockSpec((1, _TN, c), lambda bi, i: (bi, i, 0)),
            pl.BlockSpec((1, _TN, 3), lambda bi, i: (bi, i, 0)),
            wfull(Wz.shape), wfull(Wr.shape), wfull(Ws.shape),
            wfull(bz.shape), wfull(br.shape), wfull(bs.shape),
        ],
        out_specs=[
            pl.BlockSpec((1, _TN, 3 * c), lambda bi, i: (bi, i, 0)),
            pl.BlockSpec((1, _TN, 3 * c), lambda bi, i: (bi, i, 0)),
        ],
        out_shape=[
            jax.ShapeDtypeStruct((b, n, 3 * c), jnp.float32),
            jax.ShapeDtypeStruct((b, n, 3 * c), jnp.float32),
        ],
    )(phT, cfT, point, Wz, Wr, Ws, bz, br, bs)

    sc = _make_sc_gather_max(b * n, 3 * c, k)
    gmax = sc(u.reshape(b * n, 3 * c), idx.reshape(b * n, k))
    gmax = gmax.reshape(b, n, 3 * c)

    s1t = pl.pallas_call(
        _final_body,
        grid=grid,
        in_specs=[
            pl.BlockSpec((1, _TN, 3 * c), lambda bi, i: (bi, i, 0)),
            pl.BlockSpec((1, _TN, 3 * c), lambda bi, i: (bi, i, 0)),
            pl.BlockSpec((1, _TN, c), lambda bi, i: (bi, i, 0)),
            pl.BlockSpec((1, _TN, c), lambda bi, i: (bi, i, 0)),
            wfull(Wfc.shape), wfull(bfc.shape),
        ],
        out_specs=pl.BlockSpec((1, _TN, c), lambda bi, i: (bi, i, 0)),
        out_shape=jax.ShapeDtypeStruct((b, n, c), jnp.float32),
    )(gmax, v, phT, cfT, Wfc, bfc)

    s1 = jnp.transpose(s1t, (0, 2, 1))
    return (s1, s1)
